# Initial kernel scaffold; baseline (speedup 1.0000x reference)
import jax
import jax.numpy as jnp
from jax.experimental import pallas as pl


def kernel(features_0, features_1, features_2, edge_index_0, edge_index_1, edge_index_2, W_self_0, W_neigh_0, b_0, W_self_1, W_neigh_1, b_1, W_fc, b_fc, W_ih, W_hh, b_ih, b_hh):
    raise NotImplementedError("write your pallas kernel here")


# speedup vs baseline: 2.0630x; 2.0630x over previous
import functools

import jax
import jax.numpy as jnp
from jax import lax
from jax.experimental import pallas as pl
from jax.experimental.pallas import tpu as pltpu
from jax.experimental.pallas import tpu_sc as plsc

N = 10000
E = 160000
D = 256
H = 256
O = 128

NS = 16
NPAD = 10016
RPT = NPAD // NS
WB_STEPS = [(0, 128), (128, 128), (256, 128), (384, 128), (512, RPT - 512)]
TRASH = N

CHUNK = 128
CHUNKS = 80
EPAD = NS * CHUNKS * CHUNK
EROWS = EPAD // CHUNK

BLK = 400
GRID = N // BLK


def _make_sc_agg(with_deg):
    mesh = plsc.VectorSubcoreMesh(core_axis_name="c", subcore_axis_name="s")

    out_type = [jax.ShapeDtypeStruct((2 * NPAD, 128), jnp.float32)]
    scratch = [
        pltpu.VMEM_SHARED((NPAD, 128), jnp.float32),
        pltpu.VMEM((CHUNKS, 128), jnp.int32),
        pltpu.VMEM((CHUNKS, 128), jnp.int32),
        pltpu.VMEM((CHUNK, 128), jnp.float32),
        pltpu.SemaphoreType.DMA,
    ]
    if with_deg:
        out_type.append(jax.ShapeDtypeStruct((2 * NPAD, 16), jnp.float32))
        scratch += [
            pltpu.VMEM_SHARED((NPAD, 16), jnp.float32),
            pltpu.VMEM((CHUNK, 16), jnp.float32),
        ]

    def body(*refs):
        if with_deg:
            (table, src2, dst2, z128, z16, ones16,
             out_agg, out_deg,
             acc, srcv, dstv, rows, sem, dacc, dbuf) = refs
        else:
            (table, src2, dst2, z128,
             out_agg,
             acc, srcv, dstv, rows, sem) = refs

        cid = lax.axis_index("c")
        tid = lax.axis_index("s")
        r0 = tid * RPT

        pltpu.sync_copy(z128, rows)
        for off, sz in WB_STEPS:
            pltpu.sync_copy(rows.at[pl.ds(0, sz)], acc.at[pl.ds(r0 + off, sz)])
        if with_deg:
            pltpu.sync_copy(z16, dbuf)
            for off, sz in WB_STEPS:
                pltpu.sync_copy(dbuf.at[pl.ds(0, sz)],
                                dacc.at[pl.ds(r0 + off, sz)])
            pltpu.sync_copy(ones16, dbuf)

        pltpu.sync_copy(src2.at[pl.ds(cid * EROWS + tid * CHUNKS, CHUNKS)], srcv)
        pltpu.sync_copy(dst2.at[pl.ds(tid * CHUNKS, CHUNKS)], dstv)
        plsc.subcore_barrier()

        def chunk(k, carry):
            pltpu.async_copy(table.at[srcv.at[k]], rows, sem).wait()
            pltpu.sync_copy(rows, acc.at[dstv.at[k]], add=True)
            if with_deg:
                pltpu.sync_copy(dbuf, dacc.at[dstv.at[k]], add=True)
            return carry

        lax.fori_loop(0, CHUNKS, chunk, 0)
        plsc.subcore_barrier()

        o0 = cid * NPAD + r0
        for off, sz in WB_STEPS:
            pltpu.sync_copy(acc.at[pl.ds(r0 + off, sz)], rows.at[pl.ds(0, sz)])
            pltpu.sync_copy(rows.at[pl.ds(0, sz)], out_agg.at[pl.ds(o0 + off, sz)])
        if with_deg:
            for off, sz in WB_STEPS:
                pltpu.sync_copy(dacc.at[pl.ds(r0 + off, sz)],
                                dbuf.at[pl.ds(0, sz)])
                pltpu.sync_copy(dbuf.at[pl.ds(0, sz)],
                                out_deg.at[pl.ds(o0 + off, sz)])

    return pl.kernel(body, out_type=out_type, mesh=mesh, scratch_types=scratch,
                     compiler_params=pltpu.CompilerParams(
                         use_tc_tiling_on_sc=False))


def _tc_layer0(x, agg, deg, ws, wn, b):
    def body(x_ref, al_ref, ah_ref, deg_ref, ws_ref, wn_ref, b_ref, out_ref):
        rdeg = 1.0 / jnp.maximum(deg_ref[0][:, :1], 1.0)
        al = al_ref[0] * rdeg
        ah = ah_ref[0] * rdeg
        acc = jnp.dot(x_ref[...], ws_ref[...], preferred_element_type=jnp.float32)
        acc += jnp.dot(al, wn_ref[:128, :], preferred_element_type=jnp.float32)
        acc += jnp.dot(ah, wn_ref[128:, :], preferred_element_type=jnp.float32)
        h = jnp.maximum(acc + b_ref[...], 0.0)
        out_ref[0] = h[:, :128]
        out_ref[1] = h[:, 128:]

    return pl.pallas_call(
        body,
        grid=(GRID,),
        in_specs=[
            pl.BlockSpec((BLK, D), lambda i: (i, 0)),
            pl.BlockSpec((1, BLK, 128), lambda i: (0, i, 0)),
            pl.BlockSpec((1, BLK, 128), lambda i: (1, i, 0)),
            pl.BlockSpec((1, BLK, 16), lambda i: (0, i, 0)),
            pl.BlockSpec((D, H), lambda i: (0, 0)),
            pl.BlockSpec((D, H), lambda i: (0, 0)),
            pl.BlockSpec((1, H), lambda i: (0, 0)),
        ],
        out_specs=pl.BlockSpec((2, BLK, 128), lambda i: (0, i, 0)),
        out_shape=jax.ShapeDtypeStruct((2, N, 128), jnp.float32),
    )(x, agg, agg, deg, ws, wn, b)


def _tc_layer1(h1s, agg, deg, ws, wn, wfc, b1, bfc):
    def body(h1_ref, al_ref, ah_ref, deg_ref, ws_ref, wn_ref, wfc_ref,
             b1_ref, bfc_ref, out_ref):
        rdeg = 1.0 / jnp.maximum(deg_ref[0][:, :1], 1.0)
        al = al_ref[0] * rdeg
        ah = ah_ref[0] * rdeg
        h1l = h1_ref[0]
        h1h = h1_ref[1]
        acc = jnp.dot(h1l, ws_ref[:128, :], preferred_element_type=jnp.float32)
        acc += jnp.dot(h1h, ws_ref[128:, :], preferred_element_type=jnp.float32)
        acc += jnp.dot(al, wn_ref[:128, :], preferred_element_type=jnp.float32)
        acc += jnp.dot(ah, wn_ref[128:, :], preferred_element_type=jnp.float32)
        h2 = jnp.maximum(acc + b1_ref[...], 0.0)
        out_ref[...] = jnp.dot(h2, wfc_ref[...],
                               preferred_element_type=jnp.float32) + bfc_ref[...]

    return pl.pallas_call(
        body,
        grid=(GRID,),
        in_specs=[
            pl.BlockSpec((2, BLK, 128), lambda i: (0, i, 0)),
            pl.BlockSpec((1, BLK, 128), lambda i: (0, i, 0)),
            pl.BlockSpec((1, BLK, 128), lambda i: (1, i, 0)),
            pl.BlockSpec((1, BLK, 16), lambda i: (0, i, 0)),
            pl.BlockSpec((H, H), lambda i: (0, 0)),
            pl.BlockSpec((H, H), lambda i: (0, 0)),
            pl.BlockSpec((H, O), lambda i: (0, 0)),
            pl.BlockSpec((1, H), lambda i: (0, 0)),
            pl.BlockSpec((1, O), lambda i: (0, 0)),
        ],
        out_specs=pl.BlockSpec((BLK, O), lambda i: (i, 0)),
        out_shape=jax.ShapeDtypeStruct((N, O), jnp.float32),
    )(h1s, agg, agg, deg, ws, wn, wfc, b1, bfc)


def _tc_gru(y0, y1, y2, wihT, whhT, bih, bhh):
    def body(y0_ref, y1_ref, y2_ref, wih_ref, whh_ref, bih_ref, bhh_ref, out_ref):
        h = jnp.zeros((BLK, H), jnp.float32)
        for y_ref in (y0_ref, y1_ref, y2_ref):
            gi = jnp.dot(y_ref[...], wih_ref[...],
                         preferred_element_type=jnp.float32) + bih_ref[...]
            gh = jnp.dot(h, whh_ref[...],
                         preferred_element_type=jnp.float32) + bhh_ref[...]
            r = jax.nn.sigmoid(gi[:, :H] + gh[:, :H])
            z = jax.nn.sigmoid(gi[:, H:2 * H] + gh[:, H:2 * H])
            n = jnp.tanh(gi[:, 2 * H:] + r * gh[:, 2 * H:])
            h = (1.0 - z) * n + z * h
        out_ref[...] = h

    return pl.pallas_call(
        body,
        grid=(GRID,),
        in_specs=[
            pl.BlockSpec((BLK, O), lambda i: (i, 0)),
            pl.BlockSpec((BLK, O), lambda i: (i, 0)),
            pl.BlockSpec((BLK, O), lambda i: (i, 0)),
            pl.BlockSpec((O, 3 * H), lambda i: (0, 0)),
            pl.BlockSpec((H, 3 * H), lambda i: (0, 0)),
            pl.BlockSpec((1, 3 * H), lambda i: (0, 0)),
            pl.BlockSpec((1, 3 * H), lambda i: (0, 0)),
        ],
        out_specs=pl.BlockSpec((BLK, H), lambda i: (i, 0)),
        out_shape=jax.ShapeDtypeStruct((N, H), jnp.float32),
    )(y0, y1, y2, wihT, whhT, bih, bhh)


def kernel(features_0, features_1, features_2,
           edge_index_0, edge_index_1, edge_index_2,
           W_self_0, W_neigh_0, b_0, W_self_1, W_neigh_1, b_1, W_fc, b_fc,
           W_ih, W_hh, b_ih, b_hh):
    sc_agg_deg = _make_sc_agg(True)
    sc_agg = _make_sc_agg(False)

    z128 = jnp.zeros((CHUNK, 128), jnp.float32)
    z16 = jnp.zeros((CHUNK, 16), jnp.float32)
    ones16 = jnp.ones((CHUNK, 16), jnp.float32)

    b0r = b_0.reshape(1, H)
    b1r = b_1.reshape(1, H)
    bfcr = b_fc.reshape(1, O)
    wihT = W_ih.T
    whhT = W_hh.T
    bihr = b_ih.reshape(1, 3 * H)
    bhhr = b_hh.reshape(1, 3 * H)

    ys = []
    for feats, ei in ((features_0, edge_index_0),
                      (features_1, edge_index_1),
                      (features_2, edge_index_2)):
        src = ei[0]
        dst = ei[1]
        src_p = jnp.concatenate([src, jnp.zeros((EPAD - E,), jnp.int32)])
        dst_p = jnp.concatenate([dst, jnp.full((EPAD - E,), TRASH, jnp.int32)])
        src2 = jnp.concatenate([src_p, src_p + N]).reshape(2 * EROWS, 128)
        dst2 = dst_p.reshape(EROWS, 128)

        table0 = jnp.concatenate([feats[:, :128], feats[:, 128:]], axis=0)
        agg0, deg = sc_agg_deg(table0, src2, dst2, z128, z16, ones16)
        agg0 = agg0.reshape(2, NPAD, 128)
        deg = deg.reshape(2, NPAD, 16)
        h1s = _tc_layer0(feats, agg0, deg, W_self_0, W_neigh_0, b0r)
        (agg1,) = sc_agg(h1s.reshape(2 * N, 128), src2, dst2, z128)
        agg1 = agg1.reshape(2, NPAD, 128)
        y = _tc_layer1(h1s, agg1, deg, W_self_1, W_neigh_1, W_fc, b1r, bfcr)
        ys.append(y)

    final = _tc_gru(ys[0], ys[1], ys[2], wihT, whhT, bihr, bhhr)
    yearly = jnp.stack(ys, axis=1)
    return final, yearly


# speedup vs baseline: 2.2834x; 1.1068x over previous
import functools

import jax
import jax.numpy as jnp
from jax import lax
from jax.experimental import pallas as pl
from jax.experimental.pallas import tpu as pltpu
from jax.experimental.pallas import tpu_sc as plsc

N = 10000
E = 160000
D = 256
H = 256
O = 128

NS = 16
NPAD = 10016
RPT = NPAD // NS
WB_STEPS = [(0, 128), (128, 128), (256, 128), (384, 128), (512, RPT - 512)]
TRASH = N

CHUNK = 128
CHUNKS = 80
GCH = 16
GROUPS = CHUNKS // GCH
EPAD = NS * CHUNKS * CHUNK
EROWS = EPAD // CHUNK

BLK = 400
GRID = N // BLK


def _make_sc_agg(with_deg):
    mesh = plsc.VectorSubcoreMesh(core_axis_name="c", subcore_axis_name="s")

    out_type = [jax.ShapeDtypeStruct((2 * NPAD, 128), jnp.float32)]
    scratch = [
        pltpu.VMEM_SHARED((NPAD, 128), jnp.float32),
        pltpu.VMEM((GCH, 128), jnp.int32),
        pltpu.VMEM((GCH, 128), jnp.int32),
        pltpu.VMEM((CHUNK, 128), jnp.float32),
        pltpu.VMEM((CHUNK, 128), jnp.float32),
        pltpu.SemaphoreType.DMA,
        pltpu.SemaphoreType.DMA,
        pltpu.SemaphoreType.DMA,
        pltpu.SemaphoreType.DMA,
        pltpu.SemaphoreType.DMA,
    ]
    if with_deg:
        out_type.append(jax.ShapeDtypeStruct((2 * NPAD, 16), jnp.float32))
        scratch += [
            pltpu.VMEM_SHARED((NPAD, 16), jnp.float32),
            pltpu.VMEM((CHUNK, 16), jnp.float32),
        ]

    def body(*refs):
        if with_deg:
            (table, src2, dst2, z128, z16, ones16,
             out_agg, out_deg,
             acc, srcv, dstv, rows0, rows1,
             gsem0, gsem1, ssem0, ssem1, dsem, dacc, dbuf) = refs
        else:
            (table, src2, dst2, z128,
             out_agg,
             acc, srcv, dstv, rows0, rows1,
             gsem0, gsem1, ssem0, ssem1, dsem) = refs
        rows = rows0
        rbufs = (rows0, rows1)
        gsems = (gsem0, gsem1)
        ssems = (ssem0, ssem1)

        cid = lax.axis_index("c")
        tid = lax.axis_index("s")
        r0 = tid * RPT

        pltpu.sync_copy(z128, rows)
        for off, sz in WB_STEPS:
            pltpu.sync_copy(rows.at[pl.ds(0, sz)], acc.at[pl.ds(r0 + off, sz)])
        if with_deg:
            pltpu.sync_copy(z16, dbuf)
            for off, sz in WB_STEPS:
                pltpu.sync_copy(dbuf.at[pl.ds(0, sz)],
                                dacc.at[pl.ds(r0 + off, sz)])
            pltpu.sync_copy(ones16, dbuf)

        plsc.subcore_barrier()

        def group(g, carry):
            pltpu.sync_copy(
                src2.at[pl.ds(cid * EROWS + tid * CHUNKS + g * GCH, GCH)], srcv)
            pltpu.sync_copy(dst2.at[pl.ds(tid * CHUNKS + g * GCH, GCH)], dstv)
            gd = [None, None]
            sd = [None, None]
            dd = None
            gd[0] = pltpu.async_copy(table.at[srcv.at[0]], rbufs[0], gsems[0])
            for j in range(GCH):
                b = j & 1
                gd[b].wait()
                if j + 1 < GCH:
                    if sd[1 - b] is not None:
                        sd[1 - b].wait()
                    gd[1 - b] = pltpu.async_copy(
                        table.at[srcv.at[j + 1]], rbufs[1 - b], gsems[1 - b])
                sd[b] = pltpu.async_copy(
                    rbufs[b], acc.at[dstv.at[j]], ssems[b], add=True)
                if with_deg:
                    if dd is not None:
                        dd.wait()
                    dd = pltpu.async_copy(dbuf, dacc.at[dstv.at[j]], dsem,
                                          add=True)
            sd[0].wait()
            sd[1].wait()
            if with_deg:
                dd.wait()
            return carry

        lax.fori_loop(0, GROUPS, group, 0)
        plsc.subcore_barrier()

        o0 = cid * NPAD + r0
        for off, sz in WB_STEPS:
            pltpu.sync_copy(acc.at[pl.ds(r0 + off, sz)], rows.at[pl.ds(0, sz)])
            pltpu.sync_copy(rows.at[pl.ds(0, sz)], out_agg.at[pl.ds(o0 + off, sz)])
        if with_deg:
            for off, sz in WB_STEPS:
                pltpu.sync_copy(dacc.at[pl.ds(r0 + off, sz)],
                                dbuf.at[pl.ds(0, sz)])
                pltpu.sync_copy(dbuf.at[pl.ds(0, sz)],
                                out_deg.at[pl.ds(o0 + off, sz)])

    return pl.kernel(body, out_type=out_type, mesh=mesh, scratch_types=scratch,
                     compiler_params=pltpu.CompilerParams(
                         use_tc_tiling_on_sc=False))


def _tc_layer0(x, agg, deg, ws, wn, b):
    def body(x_ref, al_ref, ah_ref, deg_ref, ws_ref, wn_ref, b_ref, out_ref):
        rdeg = 1.0 / jnp.maximum(deg_ref[0][:, :1], 1.0)
        al = al_ref[0] * rdeg
        ah = ah_ref[0] * rdeg
        acc = jnp.dot(x_ref[...], ws_ref[...], preferred_element_type=jnp.float32)
        acc += jnp.dot(al, wn_ref[:128, :], preferred_element_type=jnp.float32)
        acc += jnp.dot(ah, wn_ref[128:, :], preferred_element_type=jnp.float32)
        h = jnp.maximum(acc + b_ref[...], 0.0)
        out_ref[0] = h[:, :128]
        out_ref[1] = h[:, 128:]

    return pl.pallas_call(
        body,
        grid=(GRID,),
        in_specs=[
            pl.BlockSpec((BLK, D), lambda i: (i, 0)),
            pl.BlockSpec((1, BLK, 128), lambda i: (0, i, 0)),
            pl.BlockSpec((1, BLK, 128), lambda i: (1, i, 0)),
            pl.BlockSpec((1, BLK, 16), lambda i: (0, i, 0)),
            pl.BlockSpec((D, H), lambda i: (0, 0)),
            pl.BlockSpec((D, H), lambda i: (0, 0)),
            pl.BlockSpec((1, H), lambda i: (0, 0)),
        ],
        out_specs=pl.BlockSpec((2, BLK, 128), lambda i: (0, i, 0)),
        out_shape=jax.ShapeDtypeStruct((2, N, 128), jnp.float32),
    )(x, agg, agg, deg, ws, wn, b)


def _tc_layer1(h1s, agg, deg, ws, wn, wfc, b1, bfc):
    def body(h1_ref, al_ref, ah_ref, deg_ref, ws_ref, wn_ref, wfc_ref,
             b1_ref, bfc_ref, out_ref):
        rdeg = 1.0 / jnp.maximum(deg_ref[0][:, :1], 1.0)
        al = al_ref[0] * rdeg
        ah = ah_ref[0] * rdeg
        h1l = h1_ref[0]
        h1h = h1_ref[1]
        acc = jnp.dot(h1l, ws_ref[:128, :], preferred_element_type=jnp.float32)
        acc += jnp.dot(h1h, ws_ref[128:, :], preferred_element_type=jnp.float32)
        acc += jnp.dot(al, wn_ref[:128, :], preferred_element_type=jnp.float32)
        acc += jnp.dot(ah, wn_ref[128:, :], preferred_element_type=jnp.float32)
        h2 = jnp.maximum(acc + b1_ref[...], 0.0)
        out_ref[...] = jnp.dot(h2, wfc_ref[...],
                               preferred_element_type=jnp.float32) + bfc_ref[...]

    return pl.pallas_call(
        body,
        grid=(GRID,),
        in_specs=[
            pl.BlockSpec((2, BLK, 128), lambda i: (0, i, 0)),
            pl.BlockSpec((1, BLK, 128), lambda i: (0, i, 0)),
            pl.BlockSpec((1, BLK, 128), lambda i: (1, i, 0)),
            pl.BlockSpec((1, BLK, 16), lambda i: (0, i, 0)),
            pl.BlockSpec((H, H), lambda i: (0, 0)),
            pl.BlockSpec((H, H), lambda i: (0, 0)),
            pl.BlockSpec((H, O), lambda i: (0, 0)),
            pl.BlockSpec((1, H), lambda i: (0, 0)),
            pl.BlockSpec((1, O), lambda i: (0, 0)),
        ],
        out_specs=pl.BlockSpec((BLK, O), lambda i: (i, 0)),
        out_shape=jax.ShapeDtypeStruct((N, O), jnp.float32),
    )(h1s, agg, agg, deg, ws, wn, wfc, b1, bfc)


def _tc_gru(y0, y1, y2, wihT, whhT, bih, bhh):
    def body(y0_ref, y1_ref, y2_ref, wih_ref, whh_ref, bih_ref, bhh_ref, out_ref):
        h = jnp.zeros((BLK, H), jnp.float32)
        for y_ref in (y0_ref, y1_ref, y2_ref):
            gi = jnp.dot(y_ref[...], wih_ref[...],
                         preferred_element_type=jnp.float32) + bih_ref[...]
            gh = jnp.dot(h, whh_ref[...],
                         preferred_element_type=jnp.float32) + bhh_ref[...]
            r = jax.nn.sigmoid(gi[:, :H] + gh[:, :H])
            z = jax.nn.sigmoid(gi[:, H:2 * H] + gh[:, H:2 * H])
            n = jnp.tanh(gi[:, 2 * H:] + r * gh[:, 2 * H:])
            h = (1.0 - z) * n + z * h
        out_ref[...] = h

    return pl.pallas_call(
        body,
        grid=(GRID,),
        in_specs=[
            pl.BlockSpec((BLK, O), lambda i: (i, 0)),
            pl.BlockSpec((BLK, O), lambda i: (i, 0)),
            pl.BlockSpec((BLK, O), lambda i: (i, 0)),
            pl.BlockSpec((O, 3 * H), lambda i: (0, 0)),
            pl.BlockSpec((H, 3 * H), lambda i: (0, 0)),
            pl.BlockSpec((1, 3 * H), lambda i: (0, 0)),
            pl.BlockSpec((1, 3 * H), lambda i: (0, 0)),
        ],
        out_specs=pl.BlockSpec((BLK, H), lambda i: (i, 0)),
        out_shape=jax.ShapeDtypeStruct((N, H), jnp.float32),
    )(y0, y1, y2, wihT, whhT, bih, bhh)


def kernel(features_0, features_1, features_2,
           edge_index_0, edge_index_1, edge_index_2,
           W_self_0, W_neigh_0, b_0, W_self_1, W_neigh_1, b_1, W_fc, b_fc,
           W_ih, W_hh, b_ih, b_hh):
    sc_agg_deg = _make_sc_agg(True)
    sc_agg = _make_sc_agg(False)

    z128 = jnp.zeros((CHUNK, 128), jnp.float32)
    z16 = jnp.zeros((CHUNK, 16), jnp.float32)
    ones16 = jnp.ones((CHUNK, 16), jnp.float32)

    b0r = b_0.reshape(1, H)
    b1r = b_1.reshape(1, H)
    bfcr = b_fc.reshape(1, O)
    wihT = W_ih.T
    whhT = W_hh.T
    bihr = b_ih.reshape(1, 3 * H)
    bhhr = b_hh.reshape(1, 3 * H)

    ys = []
    for feats, ei in ((features_0, edge_index_0),
                      (features_1, edge_index_1),
                      (features_2, edge_index_2)):
        src = ei[0]
        dst = ei[1]
        src_p = jnp.concatenate([src, jnp.zeros((EPAD - E,), jnp.int32)])
        dst_p = jnp.concatenate([dst, jnp.full((EPAD - E,), TRASH, jnp.int32)])
        src2 = jnp.concatenate([src_p, src_p + N]).reshape(2 * EROWS, 128)
        dst2 = dst_p.reshape(EROWS, 128)

        table0 = jnp.concatenate([feats[:, :128], feats[:, 128:]], axis=0)
        agg0, deg = sc_agg_deg(table0, src2, dst2, z128, z16, ones16)
        agg0 = agg0.reshape(2, NPAD, 128)
        deg = deg.reshape(2, NPAD, 16)
        h1s = _tc_layer0(feats, agg0, deg, W_self_0, W_neigh_0, b0r)
        (agg1,) = sc_agg(h1s.reshape(2 * N, 128), src2, dst2, z128)
        agg1 = agg1.reshape(2, NPAD, 128)
        y = _tc_layer1(h1s, agg1, deg, W_self_1, W_neigh_1, W_fc, b1r, bfcr)
        ys.append(y)

    final = _tc_gru(ys[0], ys[1], ys[2], wihT, whhT, bihr, bhhr)
    yearly = jnp.stack(ys, axis=1)
    return final, yearly


# speedup vs baseline: 2.3985x; 1.0504x over previous
import functools

import jax
import jax.numpy as jnp
from jax import lax
from jax.experimental import pallas as pl
from jax.experimental.pallas import tpu as pltpu
from jax.experimental.pallas import tpu_sc as plsc

N = 10000
E = 160000
D = 256
H = 256
O = 128

NS = 16
NPAD = 10016
RPT = NPAD // NS
WB_STEPS = [(o, min(64, RPT - o)) for o in range(0, RPT, 64)]
TRASH = N

CHUNK = 64
CHUNKS = 160
GCH = 16
GROUPS = CHUNKS // GCH
NBUF = 4
EPAD = NS * CHUNKS * CHUNK
EROWS = EPAD // CHUNK

BLK = 400
GRID = N // BLK


def _make_sc_agg(with_deg):
    mesh = plsc.VectorSubcoreMesh(core_axis_name="c", subcore_axis_name="s")

    out_type = [jax.ShapeDtypeStruct((2 * NPAD, 128), jnp.float32)]
    scratch = [
        pltpu.VMEM_SHARED((NPAD, 128), jnp.float32),
        pltpu.VMEM((GCH, CHUNK), jnp.int32),
        pltpu.VMEM((GCH, CHUNK), jnp.int32),
    ] + [pltpu.VMEM((CHUNK, 128), jnp.float32) for _ in range(NBUF)] + [
        pltpu.SemaphoreType.DMA for _ in range(2 * NBUF + 1)
    ]
    if with_deg:
        out_type.append(jax.ShapeDtypeStruct((2 * NPAD, 16), jnp.float32))
        scratch += [
            pltpu.VMEM_SHARED((NPAD, 16), jnp.float32),
            pltpu.VMEM((CHUNK, 16), jnp.float32),
        ]

    def body(*refs):
        if with_deg:
            (table, src2, dst2, z128, z16, ones16,
             out_agg, out_deg, acc, srcv, dstv) = refs[:11]
            rbufs = refs[11:11 + NBUF]
            gsems = refs[11 + NBUF:11 + 2 * NBUF]
            ssems = refs[11 + 2 * NBUF:11 + 3 * NBUF]
            dsem = refs[11 + 3 * NBUF]
            dacc, dbuf = refs[12 + 3 * NBUF], refs[13 + 3 * NBUF]
        else:
            (table, src2, dst2, z128,
             out_agg, acc, srcv, dstv) = refs[:8]
            rbufs = refs[8:8 + NBUF]
            gsems = refs[8 + NBUF:8 + 2 * NBUF]
            ssems = refs[8 + 2 * NBUF:8 + 3 * NBUF]
            dsem = refs[8 + 3 * NBUF]
        rows = rbufs[0]

        cid = lax.axis_index("c")
        tid = lax.axis_index("s")
        r0 = tid * RPT

        pltpu.sync_copy(z128, rows)
        for off, sz in WB_STEPS:
            pltpu.sync_copy(rows.at[pl.ds(0, sz)], acc.at[pl.ds(r0 + off, sz)])
        if with_deg:
            pltpu.sync_copy(z16, dbuf)
            for off, sz in WB_STEPS:
                pltpu.sync_copy(dbuf.at[pl.ds(0, sz)],
                                dacc.at[pl.ds(r0 + off, sz)])
            pltpu.sync_copy(ones16, dbuf)

        plsc.subcore_barrier()

        def group(g, carry):
            pltpu.sync_copy(
                src2.at[pl.ds(cid * EROWS + tid * CHUNKS + g * GCH, GCH)], srcv)
            pltpu.sync_copy(dst2.at[pl.ds(tid * CHUNKS + g * GCH, GCH)], dstv)
            gd = [None] * NBUF
            sd = [None] * NBUF
            dds = []
            for p in range(NBUF - 1):
                gd[p] = pltpu.async_copy(table.at[srcv.at[p]], rbufs[p],
                                         gsems[p])
            for j in range(GCH):
                b = j % NBUF
                gd[b].wait()
                nj = j + NBUF - 1
                if nj < GCH:
                    nb = nj % NBUF
                    if sd[nb] is not None:
                        sd[nb].wait()
                    gd[nb] = pltpu.async_copy(table.at[srcv.at[nj]],
                                              rbufs[nb], gsems[nb])
                sd[b] = pltpu.async_copy(
                    rbufs[b], acc.at[dstv.at[j]], ssems[b], add=True)
                if with_deg:
                    dds.append(pltpu.async_copy(dbuf, dacc.at[dstv.at[j]],
                                                dsem, add=True))
            for b in range(NBUF):
                if sd[b] is not None:
                    sd[b].wait()
            for dd in dds:
                dd.wait()
            return carry

        lax.fori_loop(0, GROUPS, group, 0)
        plsc.subcore_barrier()

        o0 = cid * NPAD + r0
        for off, sz in WB_STEPS:
            pltpu.sync_copy(acc.at[pl.ds(r0 + off, sz)], rows.at[pl.ds(0, sz)])
            pltpu.sync_copy(rows.at[pl.ds(0, sz)], out_agg.at[pl.ds(o0 + off, sz)])
        if with_deg:
            for off, sz in WB_STEPS:
                pltpu.sync_copy(dacc.at[pl.ds(r0 + off, sz)],
                                dbuf.at[pl.ds(0, sz)])
                pltpu.sync_copy(dbuf.at[pl.ds(0, sz)],
                                out_deg.at[pl.ds(o0 + off, sz)])

    return pl.kernel(body, out_type=out_type, mesh=mesh, scratch_types=scratch,
                     compiler_params=pltpu.CompilerParams(
                         use_tc_tiling_on_sc=False))


def _tc_layer0(x, agg, deg, ws, wn, b):
    def body(x_ref, al_ref, ah_ref, deg_ref, ws_ref, wn_ref, b_ref, out_ref):
        rdeg = 1.0 / jnp.maximum(deg_ref[0][:, :1], 1.0)
        al = al_ref[0] * rdeg
        ah = ah_ref[0] * rdeg
        acc = jnp.dot(x_ref[...], ws_ref[...], preferred_element_type=jnp.float32)
        acc += jnp.dot(al, wn_ref[:128, :], preferred_element_type=jnp.float32)
        acc += jnp.dot(ah, wn_ref[128:, :], preferred_element_type=jnp.float32)
        h = jnp.maximum(acc + b_ref[...], 0.0)
        out_ref[0] = h[:, :128]
        out_ref[1] = h[:, 128:]

    return pl.pallas_call(
        body,
        grid=(GRID,),
        in_specs=[
            pl.BlockSpec((BLK, D), lambda i: (i, 0)),
            pl.BlockSpec((1, BLK, 128), lambda i: (0, i, 0)),
            pl.BlockSpec((1, BLK, 128), lambda i: (1, i, 0)),
            pl.BlockSpec((1, BLK, 16), lambda i: (0, i, 0)),
            pl.BlockSpec((D, H), lambda i: (0, 0)),
            pl.BlockSpec((D, H), lambda i: (0, 0)),
            pl.BlockSpec((1, H), lambda i: (0, 0)),
        ],
        out_specs=pl.BlockSpec((2, BLK, 128), lambda i: (0, i, 0)),
        out_shape=jax.ShapeDtypeStruct((2, N, 128), jnp.float32),
    )(x, agg, agg, deg, ws, wn, b)


def _tc_layer1(h1s, agg, deg, ws, wn, wfc, b1, bfc):
    def body(h1_ref, al_ref, ah_ref, deg_ref, ws_ref, wn_ref, wfc_ref,
             b1_ref, bfc_ref, out_ref):
        rdeg = 1.0 / jnp.maximum(deg_ref[0][:, :1], 1.0)
        al = al_ref[0] * rdeg
        ah = ah_ref[0] * rdeg
        h1l = h1_ref[0]
        h1h = h1_ref[1]
        acc = jnp.dot(h1l, ws_ref[:128, :], preferred_element_type=jnp.float32)
        acc += jnp.dot(h1h, ws_ref[128:, :], preferred_element_type=jnp.float32)
        acc += jnp.dot(al, wn_ref[:128, :], preferred_element_type=jnp.float32)
        acc += jnp.dot(ah, wn_ref[128:, :], preferred_element_type=jnp.float32)
        h2 = jnp.maximum(acc + b1_ref[...], 0.0)
        out_ref[...] = jnp.dot(h2, wfc_ref[...],
                               preferred_element_type=jnp.float32) + bfc_ref[...]

    return pl.pallas_call(
        body,
        grid=(GRID,),
        in_specs=[
            pl.BlockSpec((2, BLK, 128), lambda i: (0, i, 0)),
            pl.BlockSpec((1, BLK, 128), lambda i: (0, i, 0)),
            pl.BlockSpec((1, BLK, 128), lambda i: (1, i, 0)),
            pl.BlockSpec((1, BLK, 16), lambda i: (0, i, 0)),
            pl.BlockSpec((H, H), lambda i: (0, 0)),
            pl.BlockSpec((H, H), lambda i: (0, 0)),
            pl.BlockSpec((H, O), lambda i: (0, 0)),
            pl.BlockSpec((1, H), lambda i: (0, 0)),
            pl.BlockSpec((1, O), lambda i: (0, 0)),
        ],
        out_specs=pl.BlockSpec((BLK, O), lambda i: (i, 0)),
        out_shape=jax.ShapeDtypeStruct((N, O), jnp.float32),
    )(h1s, agg, agg, deg, ws, wn, wfc, b1, bfc)


def _tc_gru(y0, y1, y2, wihT, whhT, bih, bhh):
    def body(y0_ref, y1_ref, y2_ref, wih_ref, whh_ref, bih_ref, bhh_ref, out_ref):
        h = jnp.zeros((BLK, H), jnp.float32)
        for y_ref in (y0_ref, y1_ref, y2_ref):
            gi = jnp.dot(y_ref[...], wih_ref[...],
                         preferred_element_type=jnp.float32) + bih_ref[...]
            gh = jnp.dot(h, whh_ref[...],
                         preferred_element_type=jnp.float32) + bhh_ref[...]
            r = jax.nn.sigmoid(gi[:, :H] + gh[:, :H])
            z = jax.nn.sigmoid(gi[:, H:2 * H] + gh[:, H:2 * H])
            n = jnp.tanh(gi[:, 2 * H:] + r * gh[:, 2 * H:])
            h = (1.0 - z) * n + z * h
        out_ref[...] = h

    return pl.pallas_call(
        body,
        grid=(GRID,),
        in_specs=[
            pl.BlockSpec((BLK, O), lambda i: (i, 0)),
            pl.BlockSpec((BLK, O), lambda i: (i, 0)),
            pl.BlockSpec((BLK, O), lambda i: (i, 0)),
            pl.BlockSpec((O, 3 * H), lambda i: (0, 0)),
            pl.BlockSpec((H, 3 * H), lambda i: (0, 0)),
            pl.BlockSpec((1, 3 * H), lambda i: (0, 0)),
            pl.BlockSpec((1, 3 * H), lambda i: (0, 0)),
        ],
        out_specs=pl.BlockSpec((BLK, H), lambda i: (i, 0)),
        out_shape=jax.ShapeDtypeStruct((N, H), jnp.float32),
    )(y0, y1, y2, wihT, whhT, bih, bhh)


def kernel(features_0, features_1, features_2,
           edge_index_0, edge_index_1, edge_index_2,
           W_self_0, W_neigh_0, b_0, W_self_1, W_neigh_1, b_1, W_fc, b_fc,
           W_ih, W_hh, b_ih, b_hh):
    sc_agg_deg = _make_sc_agg(True)
    sc_agg = _make_sc_agg(False)

    z128 = jnp.zeros((CHUNK, 128), jnp.float32)
    z16 = jnp.zeros((CHUNK, 16), jnp.float32)
    ones16 = jnp.ones((CHUNK, 16), jnp.float32)

    b0r = b_0.reshape(1, H)
    b1r = b_1.reshape(1, H)
    bfcr = b_fc.reshape(1, O)
    wihT = W_ih.T
    whhT = W_hh.T
    bihr = b_ih.reshape(1, 3 * H)
    bhhr = b_hh.reshape(1, 3 * H)

    ys = []
    for feats, ei in ((features_0, edge_index_0),
                      (features_1, edge_index_1),
                      (features_2, edge_index_2)):
        src = ei[0]
        dst = ei[1]
        src_p = jnp.concatenate([src, jnp.zeros((EPAD - E,), jnp.int32)])
        dst_p = jnp.concatenate([dst, jnp.full((EPAD - E,), TRASH, jnp.int32)])
        src2 = jnp.concatenate([src_p, src_p + N]).reshape(2 * EROWS, CHUNK)
        dst2 = dst_p.reshape(EROWS, CHUNK)

        table0 = jnp.concatenate([feats[:, :128], feats[:, 128:]], axis=0)
        agg0, deg = sc_agg_deg(table0, src2, dst2, z128, z16, ones16)
        agg0 = agg0.reshape(2, NPAD, 128)
        deg = deg.reshape(2, NPAD, 16)
        h1s = _tc_layer0(feats, agg0, deg, W_self_0, W_neigh_0, b0r)
        (agg1,) = sc_agg(h1s.reshape(2 * N, 128), src2, dst2, z128)
        agg1 = agg1.reshape(2, NPAD, 128)
        y = _tc_layer1(h1s, agg1, deg, W_self_1, W_neigh_1, W_fc, b1r, bfcr)
        ys.append(y)

    final = _tc_gru(ys[0], ys[1], ys[2], wihT, whhT, bihr, bhhr)
    yearly = jnp.stack(ys, axis=1)
    return final, yearly


# speedup vs baseline: 2.5097x; 1.0464x over previous
import functools

import jax
import jax.numpy as jnp
from jax import lax
from jax.experimental import pallas as pl
from jax.experimental.pallas import tpu as pltpu
from jax.experimental.pallas import tpu_sc as plsc

N = 10000
E = 160000
D = 256
H = 256
O = 128

NS = 16
NPAD = 10016
RPT = NPAD // NS
WB_STEPS = [(o, min(64, RPT - o)) for o in range(0, RPT, 64)]
TRASH = N

CHUNK = 64
CHUNKS = 160
GCH = 32
GROUPS = CHUNKS // GCH
NBUF = 4
EPAD = NS * CHUNKS * CHUNK
EROWS = EPAD // CHUNK

BLK = 400
GRID = N // BLK


def _make_sc_agg(with_deg):
    mesh = plsc.VectorSubcoreMesh(core_axis_name="c", subcore_axis_name="s")

    out_type = [jax.ShapeDtypeStruct((2 * NPAD, 128), jnp.float32)]
    scratch = [
        pltpu.VMEM_SHARED((NPAD, 128), jnp.float32),
        pltpu.VMEM((GCH, CHUNK), jnp.int32),
        pltpu.VMEM((GCH, CHUNK), jnp.int32),
    ] + [pltpu.VMEM((CHUNK, 128), jnp.float32) for _ in range(NBUF)] + [
        pltpu.SemaphoreType.DMA for _ in range(2 * NBUF + 1)
    ]
    if with_deg:
        out_type.append(jax.ShapeDtypeStruct((2 * NPAD, 16), jnp.float32))
        scratch += [
            pltpu.VMEM_SHARED((NPAD, 16), jnp.float32),
            pltpu.VMEM((CHUNK, 16), jnp.float32),
        ]

    def body(*refs):
        if with_deg:
            (table, src2, dst2, z128, z16, ones16,
             out_agg, out_deg, acc, srcv, dstv) = refs[:11]
            rbufs = refs[11:11 + NBUF]
            gsems = refs[11 + NBUF:11 + 2 * NBUF]
            ssems = refs[11 + 2 * NBUF:11 + 3 * NBUF]
            dsem = refs[11 + 3 * NBUF]
            dacc, dbuf = refs[12 + 3 * NBUF], refs[13 + 3 * NBUF]
        else:
            (table, src2, dst2, z128,
             out_agg, acc, srcv, dstv) = refs[:8]
            rbufs = refs[8:8 + NBUF]
            gsems = refs[8 + NBUF:8 + 2 * NBUF]
            ssems = refs[8 + 2 * NBUF:8 + 3 * NBUF]
            dsem = refs[8 + 3 * NBUF]
        rows = rbufs[0]

        cid = lax.axis_index("c")
        tid = lax.axis_index("s")
        r0 = tid * RPT

        pltpu.sync_copy(z128, rows)
        for off, sz in WB_STEPS:
            pltpu.sync_copy(rows.at[pl.ds(0, sz)], acc.at[pl.ds(r0 + off, sz)])
        if with_deg:
            pltpu.sync_copy(z16, dbuf)
            for off, sz in WB_STEPS:
                pltpu.sync_copy(dbuf.at[pl.ds(0, sz)],
                                dacc.at[pl.ds(r0 + off, sz)])
            pltpu.sync_copy(ones16, dbuf)

        plsc.subcore_barrier()

        def group(g, carry):
            pltpu.sync_copy(
                src2.at[pl.ds(cid * EROWS + tid * CHUNKS + g * GCH, GCH)], srcv)
            pltpu.sync_copy(dst2.at[pl.ds(tid * CHUNKS + g * GCH, GCH)], dstv)
            gd = [None] * NBUF
            sd = [None] * NBUF
            dds = []
            for p in range(NBUF - 1):
                gd[p] = pltpu.async_copy(table.at[srcv.at[p]], rbufs[p],
                                         gsems[p])
            for j in range(GCH):
                b = j % NBUF
                gd[b].wait()
                nj = j + NBUF - 1
                if nj < GCH:
                    nb = nj % NBUF
                    if sd[nb] is not None:
                        sd[nb].wait()
                    gd[nb] = pltpu.async_copy(table.at[srcv.at[nj]],
                                              rbufs[nb], gsems[nb])
                sd[b] = pltpu.async_copy(
                    rbufs[b], acc.at[dstv.at[j]], ssems[b], add=True)
                if with_deg:
                    dds.append(pltpu.async_copy(dbuf, dacc.at[dstv.at[j]],
                                                dsem, add=True))
            for b in range(NBUF):
                if sd[b] is not None:
                    sd[b].wait()
            for dd in dds:
                dd.wait()
            return carry

        lax.fori_loop(0, GROUPS, group, 0)
        plsc.subcore_barrier()

        o0 = cid * NPAD + r0
        for off, sz in WB_STEPS:
            pltpu.sync_copy(acc.at[pl.ds(r0 + off, sz)], rows.at[pl.ds(0, sz)])
            pltpu.sync_copy(rows.at[pl.ds(0, sz)], out_agg.at[pl.ds(o0 + off, sz)])
        if with_deg:
            for off, sz in WB_STEPS:
                pltpu.sync_copy(dacc.at[pl.ds(r0 + off, sz)],
                                dbuf.at[pl.ds(0, sz)])
                pltpu.sync_copy(dbuf.at[pl.ds(0, sz)],
                                out_deg.at[pl.ds(o0 + off, sz)])

    return pl.kernel(body, out_type=out_type, mesh=mesh, scratch_types=scratch,
                     compiler_params=pltpu.CompilerParams(
                         use_tc_tiling_on_sc=False))


def _tc_layer0(x, agg, deg, ws, wn, b):
    def body(x_ref, al_ref, ah_ref, deg_ref, ws_ref, wn_ref, b_ref, out_ref):
        rdeg = 1.0 / jnp.maximum(deg_ref[0][:, :1], 1.0)
        al = al_ref[0] * rdeg
        ah = ah_ref[0] * rdeg
        acc = jnp.dot(x_ref[...], ws_ref[...], preferred_element_type=jnp.float32)
        acc += jnp.dot(al, wn_ref[:128, :], preferred_element_type=jnp.float32)
        acc += jnp.dot(ah, wn_ref[128:, :], preferred_element_type=jnp.float32)
        h = jnp.maximum(acc + b_ref[...], 0.0)
        out_ref[0] = h[:, :128]
        out_ref[1] = h[:, 128:]

    return pl.pallas_call(
        body,
        grid=(GRID,),
        in_specs=[
            pl.BlockSpec((BLK, D), lambda i: (i, 0)),
            pl.BlockSpec((1, BLK, 128), lambda i: (0, i, 0)),
            pl.BlockSpec((1, BLK, 128), lambda i: (1, i, 0)),
            pl.BlockSpec((1, BLK, 16), lambda i: (0, i, 0)),
            pl.BlockSpec((D, H), lambda i: (0, 0)),
            pl.BlockSpec((D, H), lambda i: (0, 0)),
            pl.BlockSpec((1, H), lambda i: (0, 0)),
        ],
        out_specs=pl.BlockSpec((2, BLK, 128), lambda i: (0, i, 0)),
        out_shape=jax.ShapeDtypeStruct((2, N, 128), jnp.float32),
    )(x, agg, agg, deg, ws, wn, b)


def _tc_layer1(h1s, agg, deg, ws, wn, wfc, b1, bfc):
    def body(h1_ref, al_ref, ah_ref, deg_ref, ws_ref, wn_ref, wfc_ref,
             b1_ref, bfc_ref, out_ref):
        rdeg = 1.0 / jnp.maximum(deg_ref[0][:, :1], 1.0)
        al = al_ref[0] * rdeg
        ah = ah_ref[0] * rdeg
        h1l = h1_ref[0]
        h1h = h1_ref[1]
        acc = jnp.dot(h1l, ws_ref[:128, :], preferred_element_type=jnp.float32)
        acc += jnp.dot(h1h, ws_ref[128:, :], preferred_element_type=jnp.float32)
        acc += jnp.dot(al, wn_ref[:128, :], preferred_element_type=jnp.float32)
        acc += jnp.dot(ah, wn_ref[128:, :], preferred_element_type=jnp.float32)
        h2 = jnp.maximum(acc + b1_ref[...], 0.0)
        out_ref[...] = jnp.dot(h2, wfc_ref[...],
                               preferred_element_type=jnp.float32) + bfc_ref[...]

    return pl.pallas_call(
        body,
        grid=(GRID,),
        in_specs=[
            pl.BlockSpec((2, BLK, 128), lambda i: (0, i, 0)),
            pl.BlockSpec((1, BLK, 128), lambda i: (0, i, 0)),
            pl.BlockSpec((1, BLK, 128), lambda i: (1, i, 0)),
            pl.BlockSpec((1, BLK, 16), lambda i: (0, i, 0)),
            pl.BlockSpec((H, H), lambda i: (0, 0)),
            pl.BlockSpec((H, H), lambda i: (0, 0)),
            pl.BlockSpec((H, O), lambda i: (0, 0)),
            pl.BlockSpec((1, H), lambda i: (0, 0)),
            pl.BlockSpec((1, O), lambda i: (0, 0)),
        ],
        out_specs=pl.BlockSpec((BLK, O), lambda i: (i, 0)),
        out_shape=jax.ShapeDtypeStruct((N, O), jnp.float32),
    )(h1s, agg, agg, deg, ws, wn, wfc, b1, bfc)


def _tc_gru(y0, y1, y2, wihT, whhT, bih, bhh):
    def body(y0_ref, y1_ref, y2_ref, wih_ref, whh_ref, bih_ref, bhh_ref, out_ref):
        h = jnp.zeros((BLK, H), jnp.float32)
        for y_ref in (y0_ref, y1_ref, y2_ref):
            gi = jnp.dot(y_ref[...], wih_ref[...],
                         preferred_element_type=jnp.float32) + bih_ref[...]
            gh = jnp.dot(h, whh_ref[...],
                         preferred_element_type=jnp.float32) + bhh_ref[...]
            r = jax.nn.sigmoid(gi[:, :H] + gh[:, :H])
            z = jax.nn.sigmoid(gi[:, H:2 * H] + gh[:, H:2 * H])
            n = jnp.tanh(gi[:, 2 * H:] + r * gh[:, 2 * H:])
            h = (1.0 - z) * n + z * h
        out_ref[...] = h

    return pl.pallas_call(
        body,
        grid=(GRID,),
        in_specs=[
            pl.BlockSpec((BLK, O), lambda i: (i, 0)),
            pl.BlockSpec((BLK, O), lambda i: (i, 0)),
            pl.BlockSpec((BLK, O), lambda i: (i, 0)),
            pl.BlockSpec((O, 3 * H), lambda i: (0, 0)),
            pl.BlockSpec((H, 3 * H), lambda i: (0, 0)),
            pl.BlockSpec((1, 3 * H), lambda i: (0, 0)),
            pl.BlockSpec((1, 3 * H), lambda i: (0, 0)),
        ],
        out_specs=pl.BlockSpec((BLK, H), lambda i: (i, 0)),
        out_shape=jax.ShapeDtypeStruct((N, H), jnp.float32),
    )(y0, y1, y2, wihT, whhT, bih, bhh)


def kernel(features_0, features_1, features_2,
           edge_index_0, edge_index_1, edge_index_2,
           W_self_0, W_neigh_0, b_0, W_self_1, W_neigh_1, b_1, W_fc, b_fc,
           W_ih, W_hh, b_ih, b_hh):
    sc_agg_deg = _make_sc_agg(True)
    sc_agg = _make_sc_agg(False)

    z128 = jnp.zeros((CHUNK, 128), jnp.float32)
    z16 = jnp.zeros((CHUNK, 16), jnp.float32)
    ones16 = jnp.ones((CHUNK, 16), jnp.float32)

    b0r = b_0.reshape(1, H)
    b1r = b_1.reshape(1, H)
    bfcr = b_fc.reshape(1, O)
    wihT = W_ih.T
    whhT = W_hh.T
    bihr = b_ih.reshape(1, 3 * H)
    bhhr = b_hh.reshape(1, 3 * H)

    ys = []
    for feats, ei in ((features_0, edge_index_0),
                      (features_1, edge_index_1),
                      (features_2, edge_index_2)):
        src = ei[0]
        dst = ei[1]
        src_p = jnp.concatenate([src, jnp.zeros((EPAD - E,), jnp.int32)])
        dst_p = jnp.concatenate([dst, jnp.full((EPAD - E,), TRASH, jnp.int32)])
        src2 = jnp.concatenate([src_p, src_p + N]).reshape(2 * EROWS, CHUNK)
        dst2 = dst_p.reshape(EROWS, CHUNK)

        table0 = jnp.concatenate([feats[:, :128], feats[:, 128:]], axis=0)
        agg0, deg = sc_agg_deg(table0, src2, dst2, z128, z16, ones16)
        agg0 = agg0.reshape(2, NPAD, 128)
        deg = deg.reshape(2, NPAD, 16)
        h1s = _tc_layer0(feats, agg0, deg, W_self_0, W_neigh_0, b0r)
        (agg1,) = sc_agg(h1s.reshape(2 * N, 128), src2, dst2, z128)
        agg1 = agg1.reshape(2, NPAD, 128)
        y = _tc_layer1(h1s, agg1, deg, W_self_1, W_neigh_1, W_fc, b1r, bfcr)
        ys.append(y)

    final = _tc_gru(ys[0], ys[1], ys[2], wihT, whhT, bihr, bhhr)
    yearly = jnp.stack(ys, axis=1)
    return final, yearly


# speedup vs baseline: 3.7242x; 1.4839x over previous
import jax
import jax.numpy as jnp
from jax import lax
from jax.experimental import pallas as pl
from jax.experimental.pallas import tpu as pltpu
from jax.experimental.pallas import tpu_sc as plsc

N = 10000
E = 160000
D = 256
H = 256
O = 128

NS = 16
NPAD = 10016
RPT = NPAD // NS
TPT = N // NS
TRASH = N
QW = 64

CHUNK = 64
CHUNKS = 160
GCH = 32
GROUPS = CHUNKS // GCH
NBUF = 4
EPAD = NS * CHUNKS * CHUNK
EROWS = EPAD // CHUNK

STEPS_A = [(o, min(64, RPT - o)) for o in range(0, RPT, 64)]
STEPS_T = [(o, min(64, TPT - o)) for o in range(0, TPT, 64)]

BLK = 400
GRID = N // BLK


def _make_sc_agg(with_deg):
    mesh = plsc.VectorSubcoreMesh(core_axis_name="c", subcore_axis_name="s")

    out_type = [jax.ShapeDtypeStruct((4 * NPAD, QW), jnp.float32)]
    scratch = [
        pltpu.VMEM_SHARED((N, QW), jnp.float32),
        pltpu.VMEM_SHARED((NPAD, QW), jnp.float32),
        pltpu.VMEM((GCH, CHUNK), jnp.int32),
        pltpu.VMEM((GCH, CHUNK), jnp.int32),
    ] + [pltpu.VMEM((CHUNK, QW), jnp.float32) for _ in range(NBUF)] + [
        pltpu.SemaphoreType.DMA for _ in range(2 * NBUF + 1)
    ]
    if with_deg:
        out_type.append(jax.ShapeDtypeStruct((2 * NPAD, 16), jnp.float32))
        scratch += [
            pltpu.VMEM_SHARED((NPAD, 16), jnp.float32),
            pltpu.VMEM((CHUNK, 16), jnp.float32),
        ]

    def body(*refs):
        if with_deg:
            (table4, src2, dst2, z64, z16, ones16,
             out_agg, out_deg, tblsp, acc, srcv, dstv) = refs[:12]
            rbufs = refs[12:12 + NBUF]
            gsems = refs[12 + NBUF:12 + 2 * NBUF]
            ssems = refs[12 + 2 * NBUF:12 + 3 * NBUF]
            dsem = refs[12 + 3 * NBUF]
            dacc, dbuf = refs[13 + 3 * NBUF], refs[14 + 3 * NBUF]
        else:
            (table4, src2, dst2, z64,
             out_agg, tblsp, acc, srcv, dstv) = refs[:9]
            rbufs = refs[9:9 + NBUF]
            gsems = refs[9 + NBUF:9 + 2 * NBUF]
            ssems = refs[9 + 2 * NBUF:9 + 3 * NBUF]
            dsem = refs[9 + 3 * NBUF]

        cid = lax.axis_index("c")
        tid = lax.axis_index("s")
        r0 = tid * RPT
        t0 = tid * TPT

        def make_edge_group(deg_pass):
            def edge_group(g, carry):
                pltpu.sync_copy(
                    src2.at[pl.ds(tid * CHUNKS + g * GCH, GCH)], srcv)
                pltpu.sync_copy(
                    dst2.at[pl.ds(tid * CHUNKS + g * GCH, GCH)], dstv)
                gd = [None] * NBUF
                sd = [None] * NBUF
                dds = []
                for p in range(NBUF - 1):
                    gd[p] = pltpu.async_copy(tblsp.at[srcv.at[p]], rbufs[p],
                                             gsems[p])
                for j in range(GCH):
                    b = j % NBUF
                    gd[b].wait()
                    nj = j + NBUF - 1
                    if nj < GCH:
                        nb = nj % NBUF
                        if sd[nb] is not None:
                            sd[nb].wait()
                        gd[nb] = pltpu.async_copy(tblsp.at[srcv.at[nj]],
                                                  rbufs[nb], gsems[nb])
                    sd[b] = pltpu.async_copy(
                        rbufs[b], acc.at[dstv.at[j]], ssems[b], add=True)
                    if deg_pass:
                        dds.append(pltpu.async_copy(
                            dbuf, dacc.at[dstv.at[j]], dsem, add=True))
                for b in range(NBUF):
                    if sd[b] is not None:
                        sd[b].wait()
                for dd in dds:
                    dd.wait()
                return carry
            return edge_group

        for p in (0, 1):
            q = 2 * p + cid
            for off, sz in STEPS_T:
                pltpu.sync_copy(table4.at[pl.ds(q * N + t0 + off, sz)],
                                rbufs[0].at[pl.ds(0, sz)])
                pltpu.sync_copy(rbufs[0].at[pl.ds(0, sz)],
                                tblsp.at[pl.ds(t0 + off, sz)])
            pltpu.sync_copy(z64, rbufs[1])
            for off, sz in STEPS_A:
                pltpu.sync_copy(rbufs[1].at[pl.ds(0, sz)],
                                acc.at[pl.ds(r0 + off, sz)])
            if with_deg and p == 0:
                pltpu.sync_copy(z16, dbuf)
                for off, sz in STEPS_A:
                    pltpu.sync_copy(dbuf.at[pl.ds(0, sz)],
                                    dacc.at[pl.ds(r0 + off, sz)])
                pltpu.sync_copy(ones16, dbuf)
            plsc.subcore_barrier()

            lax.fori_loop(0, GROUPS, make_edge_group(with_deg and p == 0), 0)
            plsc.subcore_barrier()

            o0 = q * NPAD + r0
            for off, sz in STEPS_A:
                pltpu.sync_copy(acc.at[pl.ds(r0 + off, sz)],
                                rbufs[0].at[pl.ds(0, sz)])
                pltpu.sync_copy(rbufs[0].at[pl.ds(0, sz)],
                                out_agg.at[pl.ds(o0 + off, sz)])
            if with_deg and p == 0:
                d0 = cid * NPAD + r0
                for off, sz in STEPS_A:
                    pltpu.sync_copy(dacc.at[pl.ds(r0 + off, sz)],
                                    dbuf.at[pl.ds(0, sz)])
                    pltpu.sync_copy(dbuf.at[pl.ds(0, sz)],
                                    out_deg.at[pl.ds(d0 + off, sz)])

    return pl.kernel(body, out_type=out_type, mesh=mesh, scratch_types=scratch,
                     compiler_params=pltpu.CompilerParams(
                         use_tc_tiling_on_sc=False))


def _quarter_specs():
    return [pl.BlockSpec((1, BLK, QW), (lambda i, q=q: (q, i, 0)))
            for q in range(4)]


def _tc_layer0(x, agg, deg, ws, wn, b):
    def body(x_ref, a0, a1, a2, a3, deg_ref, ws_ref, wn_ref, b_ref, out_ref):
        rdeg = 1.0 / jnp.maximum(deg_ref[0][:, :1], 1.0)
        acc = jnp.dot(x_ref[...], ws_ref[...], preferred_element_type=jnp.float32)
        for q, aq in enumerate((a0, a1, a2, a3)):
            acc += jnp.dot(aq[0] * rdeg, wn_ref[q * QW:(q + 1) * QW, :],
                           preferred_element_type=jnp.float32)
        h = jnp.maximum(acc + b_ref[...], 0.0)
        for q in range(4):
            out_ref[q] = h[:, q * QW:(q + 1) * QW]

    return pl.pallas_call(
        body,
        grid=(GRID,),
        in_specs=[pl.BlockSpec((BLK, D), lambda i: (i, 0))]
        + _quarter_specs()
        + [
            pl.BlockSpec((1, BLK, 16), lambda i: (0, i, 0)),
            pl.BlockSpec((D, H), lambda i: (0, 0)),
            pl.BlockSpec((D, H), lambda i: (0, 0)),
            pl.BlockSpec((1, H), lambda i: (0, 0)),
        ],
        out_specs=pl.BlockSpec((4, BLK, QW), lambda i: (0, i, 0)),
        out_shape=jax.ShapeDtypeStruct((4, N, QW), jnp.float32),
    )(x, agg, agg, agg, agg, deg, ws, wn, b)


def _tc_layer1(h1s, agg, deg, ws, wn, wfc, b1, bfc):
    def body(h1_ref, a0, a1, a2, a3, deg_ref, ws_ref, wn_ref, wfc_ref,
             b1_ref, bfc_ref, out_ref):
        rdeg = 1.0 / jnp.maximum(deg_ref[0][:, :1], 1.0)
        acc = jnp.dot(h1_ref[0], ws_ref[:QW, :],
                      preferred_element_type=jnp.float32)
        for q in range(1, 4):
            acc += jnp.dot(h1_ref[q], ws_ref[q * QW:(q + 1) * QW, :],
                           preferred_element_type=jnp.float32)
        for q, aq in enumerate((a0, a1, a2, a3)):
            acc += jnp.dot(aq[0] * rdeg, wn_ref[q * QW:(q + 1) * QW, :],
                           preferred_element_type=jnp.float32)
        h2 = jnp.maximum(acc + b1_ref[...], 0.0)
        out_ref[...] = jnp.dot(h2, wfc_ref[...],
                               preferred_element_type=jnp.float32) + bfc_ref[...]

    return pl.pallas_call(
        body,
        grid=(GRID,),
        in_specs=[pl.BlockSpec((4, BLK, QW), lambda i: (0, i, 0))]
        + _quarter_specs()
        + [
            pl.BlockSpec((1, BLK, 16), lambda i: (0, i, 0)),
            pl.BlockSpec((H, H), lambda i: (0, 0)),
            pl.BlockSpec((H, H), lambda i: (0, 0)),
            pl.BlockSpec((H, O), lambda i: (0, 0)),
            pl.BlockSpec((1, H), lambda i: (0, 0)),
            pl.BlockSpec((1, O), lambda i: (0, 0)),
        ],
        out_specs=pl.BlockSpec((BLK, O), lambda i: (i, 0)),
        out_shape=jax.ShapeDtypeStruct((N, O), jnp.float32),
    )(h1s, agg, agg, agg, agg, deg, ws, wn, wfc, b1, bfc)


def _tc_gru(y0, y1, y2, wihT, whhT, bih, bhh):
    def body(y0_ref, y1_ref, y2_ref, wih_ref, whh_ref, bih_ref, bhh_ref, out_ref):
        h = jnp.zeros((BLK, H), jnp.float32)
        for y_ref in (y0_ref, y1_ref, y2_ref):
            gi = jnp.dot(y_ref[...], wih_ref[...],
                         preferred_element_type=jnp.float32) + bih_ref[...]
            gh = jnp.dot(h, whh_ref[...],
                         preferred_element_type=jnp.float32) + bhh_ref[...]
            r = jax.nn.sigmoid(gi[:, :H] + gh[:, :H])
            z = jax.nn.sigmoid(gi[:, H:2 * H] + gh[:, H:2 * H])
            n = jnp.tanh(gi[:, 2 * H:] + r * gh[:, 2 * H:])
            h = (1.0 - z) * n + z * h
        out_ref[...] = h

    return pl.pallas_call(
        body,
        grid=(GRID,),
        in_specs=[
            pl.BlockSpec((BLK, O), lambda i: (i, 0)),
            pl.BlockSpec((BLK, O), lambda i: (i, 0)),
            pl.BlockSpec((BLK, O), lambda i: (i, 0)),
            pl.BlockSpec((O, 3 * H), lambda i: (0, 0)),
            pl.BlockSpec((H, 3 * H), lambda i: (0, 0)),
            pl.BlockSpec((1, 3 * H), lambda i: (0, 0)),
            pl.BlockSpec((1, 3 * H), lambda i: (0, 0)),
        ],
        out_specs=pl.BlockSpec((BLK, H), lambda i: (i, 0)),
        out_shape=jax.ShapeDtypeStruct((N, H), jnp.float32),
    )(y0, y1, y2, wihT, whhT, bih, bhh)


def kernel(features_0, features_1, features_2,
           edge_index_0, edge_index_1, edge_index_2,
           W_self_0, W_neigh_0, b_0, W_self_1, W_neigh_1, b_1, W_fc, b_fc,
           W_ih, W_hh, b_ih, b_hh):
    sc_agg_deg = _make_sc_agg(True)
    sc_agg = _make_sc_agg(False)

    z64 = jnp.zeros((CHUNK, QW), jnp.float32)
    z16 = jnp.zeros((CHUNK, 16), jnp.float32)
    ones16 = jnp.ones((CHUNK, 16), jnp.float32)

    b0r = b_0.reshape(1, H)
    b1r = b_1.reshape(1, H)
    bfcr = b_fc.reshape(1, O)
    wihT = W_ih.T
    whhT = W_hh.T
    bihr = b_ih.reshape(1, 3 * H)
    bhhr = b_hh.reshape(1, 3 * H)

    ys = []
    for feats, ei in ((features_0, edge_index_0),
                      (features_1, edge_index_1),
                      (features_2, edge_index_2)):
        src = ei[0]
        dst = ei[1]
        src_p = jnp.concatenate([src, jnp.zeros((EPAD - E,), jnp.int32)])
        dst_p = jnp.concatenate([dst, jnp.full((EPAD - E,), TRASH, jnp.int32)])
        src2 = src_p.reshape(EROWS, CHUNK)
        dst2 = dst_p.reshape(EROWS, CHUNK)

        table0 = jnp.concatenate(
            [feats[:, q * QW:(q + 1) * QW] for q in range(4)], axis=0)
        agg0, deg = sc_agg_deg(table0, src2, dst2, z64, z16, ones16)
        agg0 = agg0.reshape(4, NPAD, QW)
        deg = deg.reshape(2, NPAD, 16)
        h1s = _tc_layer0(feats, agg0, deg, W_self_0, W_neigh_0, b0r)
        (agg1,) = sc_agg(h1s.reshape(4 * N, QW), src2, dst2, z64)
        agg1 = agg1.reshape(4, NPAD, QW)
        y = _tc_layer1(h1s, agg1, deg, W_self_1, W_neigh_1, W_fc, b1r, bfcr)
        ys.append(y)

    final = _tc_gru(ys[0], ys[1], ys[2], wihT, whhT, bihr, bhhr)
    yearly = jnp.stack(ys, axis=1)
    return final, yearly
